# Initial kernel scaffold; baseline (speedup 1.0000x reference)
import jax
import jax.numpy as jnp
from jax.experimental import pallas as pl


def kernel(h, edge_index, W, b):
    raise NotImplementedError("write your pallas kernel here")


# speedup vs baseline: 5.7566x; 5.7566x over previous
import functools

import jax
import jax.numpy as jnp
from jax import lax
from jax.experimental import pallas as pl
from jax.experimental.pallas import tpu as pltpu
from jax.experimental.pallas import tpu_sc as plsc

NUM_CORES = 2
NUM_SUBCORES = 16


def _make_sc_aggregate(n, npad, d, e):
    nw = NUM_CORES * NUM_SUBCORES
    epw = e // nw
    k = 80
    nch = epw // k
    rpt = npad // NUM_SUBCORES
    mesh = plsc.VectorSubcoreMesh(
        core_axis_name="c", subcore_axis_name="s",
        num_cores=NUM_CORES, num_subcores=NUM_SUBCORES)

    @functools.partial(
        pl.kernel,
        mesh=mesh,
        out_type=jax.ShapeDtypeStruct((NUM_CORES * npad, d), jnp.float32),
        scratch_types=[
            pltpu.VMEM((k,), jnp.int32),
            pltpu.VMEM((k,), jnp.int32),
            pltpu.VMEM((k, d), jnp.float32),
            pltpu.VMEM_SHARED((npad, d), jnp.float32),
            pltpu.SemaphoreType.DMA,
        ],
    )
    def sc_agg(h_hbm, src_hbm, dst_hbm, z2_hbm,
               sum_hbm,
               src_v, dst_v, rows_v, acc_sp, sem):
        cid = lax.axis_index("c")
        sid = lax.axis_index("s")
        wid = sid * NUM_CORES + cid

        pltpu.sync_copy(z2_hbm.at[pl.ds(sid * rpt, rpt)],
                        acc_sp.at[pl.ds(sid * rpt, rpt)])
        plsc.subcore_barrier()

        ebase = wid * epw

        def chunk(i, carry):
            base = ebase + i * k
            pltpu.sync_copy(src_hbm.at[pl.ds(base, k)], src_v)
            pltpu.sync_copy(dst_hbm.at[pl.ds(base, k)], dst_v)
            pltpu.async_copy(h_hbm.at[src_v], rows_v, sem).wait()
            pltpu.sync_copy(rows_v, acc_sp.at[dst_v], add=True)
            return carry

        lax.fori_loop(0, nch, chunk, 0)
        plsc.subcore_barrier()

        pltpu.sync_copy(acc_sp.at[pl.ds(sid * rpt, rpt)],
                        sum_hbm.at[pl.ds(cid * npad + sid * rpt, rpt)])

    return sc_agg


def _deg_body(d_ref, o_ref):
    dst = d_ref[...]
    lanes = lax.broadcasted_iota(jnp.int32, (1, 128), 1)
    oh_lo = (lax.rem(dst, 128) == lanes).astype(jnp.float32)
    oh_hi = (lax.div(dst, 128) == lanes).astype(jnp.float32)
    c_blk = lax.dot_general(oh_lo, oh_hi, (((0,), (0,)), ((), ())),
                            preferred_element_type=jnp.float32)

    @pl.when(pl.program_id(0) == 0)
    def _():
        o_ref[...] = jnp.zeros_like(o_ref)

    o_ref[...] += c_blk


def _tc_degrees(dst, e):
    eb = 2560
    return pl.pallas_call(
        _deg_body,
        grid=(e // eb,),
        in_specs=[pl.BlockSpec((eb, 1), lambda i: (i, 0))],
        out_specs=pl.BlockSpec((128, 128), lambda i: (0, 0)),
        out_shape=jax.ShapeDtypeStruct((128, 128), jnp.float32),
    )(dst[:, None])


def _tc_body(h_ref, p0_ref, p1_ref, d_ref, wt_ref, b_ref, o_ref):
    d = h_ref.shape[1]
    deg = jnp.maximum(d_ref[...], 1.0)
    h_n = (p0_ref[...] + p1_ref[...]) / deg
    wt = wt_ref[...]
    o_ref[...] = (
        jnp.dot(h_ref[...], wt[:d], preferred_element_type=jnp.float32)
        + jnp.dot(h_n, wt[d:], preferred_element_type=jnp.float32)
        + b_ref[...])


def _tc_linear(h, p0, p1, deg, wt, b2):
    n, d = h.shape
    out = wt.shape[1]
    blk = 1000
    return pl.pallas_call(
        _tc_body,
        grid=(n // blk,),
        in_specs=[
            pl.BlockSpec((blk, d), lambda i: (i, 0)),
            pl.BlockSpec((blk, d), lambda i: (i, 0)),
            pl.BlockSpec((blk, d), lambda i: (i, 0)),
            pl.BlockSpec((blk, 1), lambda i: (i, 0)),
            pl.BlockSpec((2 * d, out), lambda i: (0, 0)),
            pl.BlockSpec((1, out), lambda i: (0, 0)),
        ],
        out_specs=pl.BlockSpec((blk, out), lambda i: (i, 0)),
        out_shape=jax.ShapeDtypeStruct((n, out), jnp.float32),
    )(h, p0, p1, deg, wt, b2)


def kernel(h, edge_index, W, b):
    n, d = h.shape
    e = edge_index.shape[1]
    npad = ((n + 127) // 128) * 128
    src = edge_index[0]
    dst = edge_index[1]
    z2 = jnp.zeros((npad, d), jnp.float32)
    sums = _make_sc_aggregate(n, npad, d, e)(h, src, dst, z2)
    c_mat = _tc_degrees(dst, e)
    deg = c_mat.T.reshape(-1)[:n]
    wt = W.T
    return _tc_linear(h, sums[:n], sums[npad:npad + n], deg[:, None],
                      wt, b[None, :])


# speedup vs baseline: 7.3290x; 1.2731x over previous
import functools

import jax
import jax.numpy as jnp
from jax import lax
from jax.experimental import pallas as pl
from jax.experimental.pallas import tpu as pltpu
from jax.experimental.pallas import tpu_sc as plsc

NUM_CORES = 2
NUM_SUBCORES = 16


def _make_sc_aggregate(n, npad, d, e):
    nw = NUM_CORES * NUM_SUBCORES
    epw = e // nw
    k = 80
    nch = epw // k
    rpt = npad // NUM_SUBCORES
    mesh = plsc.VectorSubcoreMesh(
        core_axis_name="c", subcore_axis_name="s",
        num_cores=NUM_CORES, num_subcores=NUM_SUBCORES)

    @functools.partial(
        pl.kernel,
        mesh=mesh,
        out_type=jax.ShapeDtypeStruct((NUM_CORES * npad, d), jnp.float32),
        scratch_types=[
            pltpu.VMEM((2, k), jnp.int32),
            pltpu.VMEM((2, k), jnp.int32),
            pltpu.VMEM((k, d), jnp.float32),
            pltpu.VMEM((k, d), jnp.float32),
            pltpu.VMEM_SHARED((npad, d), jnp.float32),
            pltpu.SemaphoreType.DMA,
            pltpu.SemaphoreType.DMA,
            pltpu.SemaphoreType.DMA,
            pltpu.SemaphoreType.DMA,
        ],
    )
    def sc_agg(h_hbm, src_hbm, dst_hbm, z2_hbm,
               sum_hbm,
               src_v, dst_v, rows_a, rows_b, acc_sp,
               sem_ia, sem_ib, sem_ga, sem_gb):
        cid = lax.axis_index("c")
        sid = lax.axis_index("s")
        wid = sid * NUM_CORES + cid

        pltpu.sync_copy(z2_hbm.at[pl.ds(sid * rpt, rpt)],
                        acc_sp.at[pl.ds(sid * rpt, rpt)])
        plsc.subcore_barrier()

        ebase = wid * epw

        def fire_idx(c, buf, sem):
            base = ebase + c * k
            d1 = pltpu.async_copy(src_hbm.at[pl.ds(base, k)],
                                  src_v.at[buf], sem)
            d2 = pltpu.async_copy(dst_hbm.at[pl.ds(base, k)],
                                  dst_v.at[buf], sem)
            return d1, d2

        def wait_idx(descs):
            descs[0].wait()
            descs[1].wait()

        def fire_gather(buf, rows, sem):
            return pltpu.async_copy(h_hbm.at[src_v.at[buf]], rows, sem)

        def scatter(buf, rows):
            pltpu.sync_copy(rows, acc_sp.at[dst_v.at[buf]], add=True)

        wait_idx(fire_idx(0, 0, sem_ia))
        fire_gather(0, rows_a, sem_ga)
        fire_idx(1, 1, sem_ib)

        def pipeline_body(j, carry):
            c = 2 * j
            pltpu.make_async_copy(src_hbm.at[pl.ds(0, k)],
                                  src_v.at[1], sem_ib).wait()
            pltpu.make_async_copy(dst_hbm.at[pl.ds(0, k)],
                                  dst_v.at[1], sem_ib).wait()
            gb = fire_gather(1, rows_b, sem_gb)
            pltpu.make_async_copy(h_hbm.at[src_v.at[0]],
                                  rows_a, sem_ga).wait()
            scatter(0, rows_a)
            wait_idx(fire_idx(c + 2, 0, sem_ia))
            fire_gather(0, rows_a, sem_ga)
            gb.wait()
            scatter(1, rows_b)
            fire_idx(c + 3, 1, sem_ib)
            return carry

        lax.fori_loop(0, (nch - 3) // 2, pipeline_body, 0)

        pltpu.make_async_copy(src_hbm.at[pl.ds(0, k)],
                              src_v.at[1], sem_ib).wait()
        pltpu.make_async_copy(dst_hbm.at[pl.ds(0, k)],
                              dst_v.at[1], sem_ib).wait()
        gb = fire_gather(1, rows_b, sem_gb)
        pltpu.make_async_copy(h_hbm.at[src_v.at[0]], rows_a, sem_ga).wait()
        scatter(0, rows_a)
        wait_idx(fire_idx(nch - 1, 0, sem_ia))
        ga = fire_gather(0, rows_a, sem_ga)
        gb.wait()
        scatter(1, rows_b)
        ga.wait()
        scatter(0, rows_a)

        plsc.subcore_barrier()

        pltpu.sync_copy(acc_sp.at[pl.ds(sid * rpt, rpt)],
                        sum_hbm.at[pl.ds(cid * npad + sid * rpt, rpt)])

    return sc_agg


def _deg_body(d_ref, o_ref):
    dst = d_ref[...]
    lanes = lax.broadcasted_iota(jnp.int32, (1, 128), 1)
    oh_lo = (lax.rem(dst, 128) == lanes).astype(jnp.float32)
    oh_hi = (lax.div(dst, 128) == lanes).astype(jnp.float32)
    c_blk = lax.dot_general(oh_lo, oh_hi, (((0,), (0,)), ((), ())),
                            preferred_element_type=jnp.float32)

    @pl.when(pl.program_id(0) == 0)
    def _():
        o_ref[...] = jnp.zeros_like(o_ref)

    o_ref[...] += c_blk


def _tc_degrees(dst, e):
    eb = 2560
    return pl.pallas_call(
        _deg_body,
        grid=(e // eb,),
        in_specs=[pl.BlockSpec((eb, 1), lambda i: (i, 0))],
        out_specs=pl.BlockSpec((128, 128), lambda i: (0, 0)),
        out_shape=jax.ShapeDtypeStruct((128, 128), jnp.float32),
    )(dst[:, None])


def _tc_body(h_ref, p0_ref, p1_ref, d_ref, wt_ref, b_ref, o_ref):
    d = h_ref.shape[1]
    deg = jnp.maximum(d_ref[...], 1.0)
    h_n = (p0_ref[...] + p1_ref[...]) / deg
    wt = wt_ref[...]
    o_ref[...] = (
        jnp.dot(h_ref[...], wt[:d], preferred_element_type=jnp.float32)
        + jnp.dot(h_n, wt[d:], preferred_element_type=jnp.float32)
        + b_ref[...])


def _tc_linear(h, p0, p1, deg, wt, b2):
    n, d = h.shape
    out = wt.shape[1]
    blk = 1000
    return pl.pallas_call(
        _tc_body,
        grid=(n // blk,),
        in_specs=[
            pl.BlockSpec((blk, d), lambda i: (i, 0)),
            pl.BlockSpec((blk, d), lambda i: (i, 0)),
            pl.BlockSpec((blk, d), lambda i: (i, 0)),
            pl.BlockSpec((blk, 1), lambda i: (i, 0)),
            pl.BlockSpec((2 * d, out), lambda i: (0, 0)),
            pl.BlockSpec((1, out), lambda i: (0, 0)),
        ],
        out_specs=pl.BlockSpec((blk, out), lambda i: (i, 0)),
        out_shape=jax.ShapeDtypeStruct((n, out), jnp.float32),
    )(h, p0, p1, deg, wt, b2)


def kernel(h, edge_index, W, b):
    n, d = h.shape
    e = edge_index.shape[1]
    npad = ((n + 127) // 128) * 128
    src = edge_index[0]
    dst = edge_index[1]
    z2 = jnp.zeros((npad, d), jnp.float32)
    sums = _make_sc_aggregate(n, npad, d, e)(h, src, dst, z2)
    c_mat = _tc_degrees(dst, e)
    deg = c_mat.T.reshape(-1)[:n]
    wt = W.T
    return _tc_linear(h, sums[:n], sums[npad:npad + n], deg[:, None],
                      wt, b[None, :])
